# baseline (speedup 1.0000x reference)
import functools

import jax
import jax.numpy as jnp
from jax import lax
from jax.experimental import pallas as pl
from jax.experimental.pallas import tpu as pltpu
from jax.experimental.pallas import tpu_sc as plsc

_NUM_TIMESTEPS = 1000
_TABLE_LEN = _NUM_TIMESTEPS + 1
_BATCH = 16384
_NC = 1
_NS = 16
_L = 16
_NW = _NC * _NS
_B_PER_W = _BATCH // _NW
_NCHUNK = 4
_CHUNK = _B_PER_W // _NCHUNK

_mesh = plsc.VectorSubcoreMesh(
    core_axis_name="c", subcore_axis_name="s", num_cores=_NC)


@functools.partial(
    pl.kernel,
    mesh=_mesh,
    out_type=jax.ShapeDtypeStruct((_BATCH,), jnp.float32),
    scratch_types=[
        pltpu.VMEM((_TABLE_LEN,), jnp.float32),
        pltpu.VMEM((_B_PER_W,), jnp.int32),
        pltpu.VMEM((_B_PER_W,), jnp.float32),
        pltpu.SemaphoreType.DMA,
        pltpu.SemaphoreType.DMA,
        pltpu.SemaphoreType.DMA,
    ],
    compiler_params=pltpu.CompilerParams(needs_layout_passes=False),
)
def _alpha_bar_gather(t_hbm, table_hbm, out_hbm, table_v, idx_v, res_v,
                      tsem, isem, osem):
    wid = lax.axis_index("s") * _NC + lax.axis_index("c")
    base = wid * _B_PER_W
    tcopy = pltpu.async_copy(table_hbm, table_v, tsem)
    icopies = [
        pltpu.async_copy(
            t_hbm.at[pl.ds(base + k * _CHUNK, _CHUNK)],
            idx_v.at[pl.ds(k * _CHUNK, _CHUNK)],
            isem,
        )
        for k in range(_NCHUNK)
    ]
    tcopy.wait()
    ocopies = []
    for k in range(_NCHUNK):
        icopies[k].wait()

        @pl.loop(k * _CHUNK // _L, (k + 1) * _CHUNK // _L, unroll=4)
        def _gather(i):
            off = i * _L
            idx = idx_v[pl.ds(off, _L)]
            idx = jnp.minimum(jnp.maximum(idx, 0), _NUM_TIMESTEPS - 1)
            res_v[pl.ds(off, _L)] = plsc.load_gather(table_v, [idx])

        ocopies.append(
            pltpu.async_copy(
                res_v.at[pl.ds(k * _CHUNK, _CHUNK)],
                out_hbm.at[pl.ds(base + k * _CHUNK, _CHUNK)],
                osem,
            )
        )
    for c in ocopies:
        c.wait()


def kernel(t, alphas_cumprod):
    return _alpha_bar_gather(t.astype(jnp.int32), alphas_cumprod)


# speedup vs baseline: 1.0038x; 1.0038x over previous
import functools

import jax
import jax.numpy as jnp
from jax import lax
from jax.experimental import pallas as pl
from jax.experimental.pallas import tpu as pltpu
from jax.experimental.pallas import tpu_sc as plsc

_NUM_TIMESTEPS = 1000
_TABLE_LEN = _NUM_TIMESTEPS + 1
_BATCH = 16384
_NC = 1
_NS = 16
_L = 16
_NW = _NC * _NS
_B_PER_W = _BATCH // _NW

_mesh = plsc.VectorSubcoreMesh(
    core_axis_name="c", subcore_axis_name="s", num_cores=_NC)


@functools.partial(
    pl.kernel,
    mesh=_mesh,
    out_type=jax.ShapeDtypeStruct((_BATCH,), jnp.float32),
    scratch_types=[
        pltpu.VMEM((_TABLE_LEN,), jnp.float32),
        pltpu.VMEM((_B_PER_W,), jnp.int32),
        pltpu.VMEM((_B_PER_W,), jnp.float32),
        pltpu.SemaphoreType.DMA,
        pltpu.SemaphoreType.DMA,
        pltpu.SemaphoreType.DMA,
    ],
    compiler_params=pltpu.CompilerParams(needs_layout_passes=False),
)
def _alpha_bar_gather(t_hbm, table_hbm, out_hbm, table_v, idx_v, res_v,
                      tsem, isem, osem):
    wid = lax.axis_index("s") * _NC + lax.axis_index("c")
    base = wid * _B_PER_W
    half = _B_PER_W // 2
    tcopy = pltpu.async_copy(table_hbm, table_v, tsem)
    icopy = pltpu.async_copy(t_hbm.at[pl.ds(base, _B_PER_W)], idx_v, isem)
    tcopy.wait()
    icopy.wait()

    @pl.loop(0, half // _L, unroll=4)
    def _first(i):
        off = i * _L
        idx = idx_v[pl.ds(off, _L)]
        idx = jnp.minimum(jnp.maximum(idx, 0), _NUM_TIMESTEPS - 1)
        res_v[pl.ds(off, _L)] = plsc.load_gather(table_v, [idx])

    ocopy0 = pltpu.async_copy(
        res_v.at[pl.ds(0, half)], out_hbm.at[pl.ds(base, half)], osem)

    @pl.loop(half // _L, _B_PER_W // _L, unroll=4)
    def _second(i):
        off = i * _L
        idx = idx_v[pl.ds(off, _L)]
        idx = jnp.minimum(jnp.maximum(idx, 0), _NUM_TIMESTEPS - 1)
        res_v[pl.ds(off, _L)] = plsc.load_gather(table_v, [idx])

    ocopy1 = pltpu.async_copy(
        res_v.at[pl.ds(half, half)], out_hbm.at[pl.ds(base + half, half)], osem)
    ocopy0.wait()
    ocopy1.wait()


def kernel(t, alphas_cumprod):
    return _alpha_bar_gather(t.astype(jnp.int32), alphas_cumprod)
